# Initial kernel scaffold; baseline (speedup 1.0000x reference)
import jax
import jax.numpy as jnp
from jax.experimental import pallas as pl


def kernel(features, laplacian_indices, laplacian_values, selfloop_indices, selfloop_values, ui_indices, ui_values, W_lin, b_lin, W_lin1, b_lin1, W_iat, b_iat, W_iat1, b_iat1):
    raise NotImplementedError("write your pallas kernel here")


# speedup vs baseline: 7.6113x; 7.6113x over previous
import functools

import jax
import jax.numpy as jnp
from jax import lax
from jax.experimental import pallas as pl
from jax.experimental.pallas import tpu as pltpu
from jax.experimental.pallas import tpu_sc as plsc

_N = 10000
_E = 320000
_D = 128

_NC = 2
_NS = 16
_NW = _NC * _NS
_L = 16

_C = 128
_E2 = 2 * _E
_PER_TILE = -(-_E2 // (_NW * _C)) * _C
_CHUNKS = _PER_TILE // _C
_EP = _PER_TILE * _NW

_NP = 10240
_RPS = _NP // _NS
_ZR = 128

_BLK = 1000


def _dense_body(f_ref, wl_ref, wi_ref, wl1_ref, wi1_ref, y_ref):
    x = f_ref[...]
    x2 = x * x
    dn = (((1,), (1,)), ((), ()))
    y_ref[0] = (lax.dot_general(x, wl_ref[...], dn, preferred_element_type=jnp.float32)
                + lax.dot_general(x2, wi_ref[...], dn, preferred_element_type=jnp.float32))
    y_ref[1] = (lax.dot_general(x, wl1_ref[...], dn, preferred_element_type=jnp.float32)
                + lax.dot_general(x2, wi1_ref[...], dn, preferred_element_type=jnp.float32))


def _dense(features, wl, wi, wl1, wi1):
    w_spec = pl.BlockSpec((_D, _D), lambda i: (0, 0))
    return pl.pallas_call(
        _dense_body,
        grid=(_N // _BLK,),
        in_specs=[pl.BlockSpec((_BLK, _D), lambda i: (i, 0))] + [w_spec] * 4,
        out_specs=pl.BlockSpec((2, _BLK, _D), lambda i: (0, i, 0)),
        out_shape=jax.ShapeDtypeStruct((2, _N, _D), jnp.float32),
    )(features, wl, wi, wl1, wi1)


def _spmm_body(y2, cols, rows, vals, out,
               colbuf, rowbuf, valbuf, gbuf, zbuf, acc, sem):
    c = lax.axis_index("c")
    s = lax.axis_index("s")
    w = c * _NS + s

    def zrow(i, carry):
        for k in range(_D // _L):
            zbuf[i, pl.ds(k * _L, _L)] = jnp.zeros((_L,), jnp.float32)
        return carry

    lax.fori_loop(0, _ZR, zrow, 0)
    for j in range(_RPS // _ZR):
        pltpu.sync_copy(zbuf, acc.at[pl.ds(s * _RPS + j * _ZR, _ZR)])
    plsc.subcore_barrier()

    base = w * _PER_TILE

    def chunk(g, carry):
        e0 = base + g * _C
        pltpu.sync_copy(cols.at[pl.ds(e0, _C)], colbuf)
        pltpu.sync_copy(rows.at[pl.ds(e0, _C)], rowbuf)
        pltpu.sync_copy(vals.at[pl.ds(e0, _C)], valbuf)
        pltpu.async_copy(y2.at[colbuf], gbuf, sem).wait()

        def grp(t, carry2):
            vvec = valbuf[pl.ds(t * _L, _L)]
            for j in range(_L):
                e = t * _L + j
                v = jnp.full((_L,), vvec[j], jnp.float32)
                for k in range(_D // _L):
                    gbuf[e, pl.ds(k * _L, _L)] = gbuf[e, pl.ds(k * _L, _L)] * v
            return carry2

        lax.fori_loop(0, _C // _L, grp, 0)
        pltpu.sync_copy(gbuf, acc.at[rowbuf], add=True)
        return carry

    lax.fori_loop(0, _CHUNKS, chunk, 0)
    plsc.subcore_barrier()

    for j in range(_RPS // _ZR):
        r0 = s * _RPS + j * _ZR
        pltpu.sync_copy(acc.at[pl.ds(r0, _ZR)], zbuf)
        pltpu.sync_copy(zbuf, out.at[c, pl.ds(r0, _ZR)])


def _spmm(y2, cols, rows, vals):
    mesh = plsc.VectorSubcoreMesh(core_axis_name="c", subcore_axis_name="s")
    return pl.kernel(
        _spmm_body,
        out_type=jax.ShapeDtypeStruct((_NC, _NP, _D), jnp.float32),
        mesh=mesh,
        scratch_types=[
            pltpu.VMEM((_C,), jnp.int32),
            pltpu.VMEM((_C,), jnp.int32),
            pltpu.VMEM((_C,), jnp.float32),
            pltpu.VMEM((_C, _D), jnp.float32),
            pltpu.VMEM((_ZR, _D), jnp.float32),
            pltpu.VMEM_SHARED((_NP, _D), jnp.float32),
            pltpu.SemaphoreType.DMA,
        ],
    )(y2, cols, rows, vals)


def _combine_body(p_ref, b_ref, o_ref):
    o_ref[...] = p_ref[0] + p_ref[1] + b_ref[...]


def _combine(partials, bias):
    return pl.pallas_call(
        _combine_body,
        grid=(_N // _BLK,),
        in_specs=[pl.BlockSpec((2, _BLK, _D), lambda i: (0, i, 0)),
                  pl.BlockSpec((1, _D), lambda i: (0, 0))],
        out_specs=pl.BlockSpec((_BLK, _D), lambda i: (i, 0)),
        out_shape=jax.ShapeDtypeStruct((_N, _D), jnp.float32),
    )(partials, bias)


def kernel(features, laplacian_indices, laplacian_values, selfloop_indices,
           selfloop_values, ui_indices, ui_values,
           W_lin, b_lin, W_lin1, b_lin1, W_iat, b_iat, W_iat1, b_iat1):
    y = _dense(features, W_lin, W_iat, W_lin1, W_iat1)
    y2 = y.reshape(2 * _N, _D)

    pad = _EP - _E2
    cols = jnp.concatenate([
        laplacian_indices[1], ui_indices[1] + _N,
        jnp.zeros((pad,), jnp.int32)])
    rows = jnp.concatenate([
        laplacian_indices[0], ui_indices[0],
        jnp.zeros((pad,), jnp.int32)])
    vals = jnp.concatenate([
        laplacian_values, ui_values, jnp.zeros((pad,), jnp.float32)])

    partials = _spmm(y2, cols, rows, vals)

    bias = (b_lin + b_iat + b_lin1 + b_iat1).reshape(1, _D)
    return _combine(partials, bias)
